# baseline (speedup 1.0000x reference)
import functools
import math

import jax
import jax.numpy as jnp
from jax import lax
from jax.experimental import pallas as pl
from jax.experimental.pallas import tpu as pltpu
from jax.experimental.pallas import tpu_sc as plsc

L = 16
NB = 256
SPARSITY = 0.05


def _kwta_sc(batch, emb, k, n_workers):
  rows_per_w = batch // n_workers
  seg = emb // L
  UNR = 8

  mesh = plsc.VectorSubcoreMesh(core_axis_name="c", subcore_axis_name="s")

  def body(x_hbm, out_hbm, xbuf, cbuf, cbuf2, hist):
    cid = lax.axis_index("c")
    sid = lax.axis_index("s")
    wid = sid * 2 + cid
    lane = lax.iota(jnp.int32, L)
    ones_i32 = jnp.ones((L,), jnp.int32)

    def ukey_of(raw_f32):
      raw = lax.bitcast_convert_type(raw_f32, jnp.uint32)
      neg = raw >= jnp.uint32(0x80000000)
      flip = jnp.where(neg, jnp.uint32(0xFFFFFFFF), jnp.uint32(0x80000000))
      return raw ^ flip

    def clear_hist():
      @plsc.parallel_loop(0, NB * L, L, unroll=UNR)
      def _(i):
        hist[pl.ds(i, L)] = jnp.zeros((L,), jnp.int32)

    def walk(k_rem):
      carry = jnp.int32(0)
      bucket_sel = jnp.int32(0)
      k_next = jnp.int32(0)
      m_sel = jnp.int32(0)
      for g in range(NB // L - 1, -1, -1):
        tg = jnp.zeros((L,), jnp.int32)
        for sub in range(L):
          gidx = g * NB + lane * L + sub
          tg = tg + plsc.load_gather(hist, [gidx])
        rc = plsc.cumsum(lax.rev(tg, (0,)))
        srev = lax.rev(rc, (0,))
        group_sum = jnp.max(rc)
        inc = srev + carry
        ge = inc >= k_rem
        cnt = jnp.max(plsc.all_reduce_population_count(ge))
        jstar = cnt - 1
        t_at = jnp.sum(jnp.where(lane == jstar, tg, 0))
        inc_at = jnp.sum(jnp.where(lane == jstar, inc, 0))
        cond = (carry < k_rem) & (carry + group_sum >= k_rem)
        bucket_sel = bucket_sel + jnp.where(cond, g * L + jstar, 0)
        k_next = k_next + jnp.where(cond, k_rem - (inc_at - t_at), 0)
        m_sel = m_sel + jnp.where(cond, t_at, 0)
        carry = carry + group_sum
      return bucket_sel, k_next, m_sel

    def do_row(r, carry_none):
      row = wid * rows_per_w + r
      pltpu.sync_copy(x_hbm.at[row], xbuf)

      clear_hist()

      @plsc.parallel_loop(0, emb, L, unroll=UNR)
      def _(i):
        uk = ukey_of(xbuf[pl.ds(i, L)])
        bucket = lax.convert_element_type(
            lax.shift_right_logical(uk, jnp.uint32(24)), jnp.int32)
        plsc.addupdate_scatter(hist, [bucket * L + lane], ones_i32)

      b0, k1, _m0 = walk(jnp.int32(k))

      clear_hist()
      b0_u = lax.convert_element_type(b0, jnp.uint32)

      @plsc.parallel_loop(0, emb, L, unroll=UNR, carry=jnp.zeros((L,), jnp.int32))
      def cnt1(i, cnt):
        uk = ukey_of(xbuf[pl.ds(i, L)])
        match = lax.shift_right_logical(uk, jnp.uint32(24)) == b0_u
        bucket = lax.convert_element_type(
            jnp.uint32(0xFF) & lax.shift_right_logical(uk, jnp.uint32(16)),
            jnp.int32)
        plsc.addupdate_scatter(hist, [bucket * L + lane], ones_i32, mask=match)
        plsc.store_scatter(cbuf, [lane * seg + cnt],
                           lax.bitcast_convert_type(uk, jnp.int32), mask=match)
        return cnt + jnp.where(match, 1, 0)

      b1, k2, _m1 = walk(k1)
      max1 = jnp.max(cnt1)
      b1_u = lax.convert_element_type(b1, jnp.uint32)

      clear_hist()

      @plsc.parallel_loop(0, max1, 1, carry=jnp.zeros((L,), jnp.int32))
      def cnt2(j, cnt):
        uk = lax.bitcast_convert_type(
            plsc.load_gather(cbuf, [lane * seg + j]), jnp.uint32)
        valid = j < cnt1
        match = valid & (
            (jnp.uint32(0xFF) & lax.shift_right_logical(uk, jnp.uint32(16)))
            == b1_u)
        bucket = lax.convert_element_type(
            jnp.uint32(0xFF) & lax.shift_right_logical(uk, jnp.uint32(8)),
            jnp.int32)
        plsc.addupdate_scatter(hist, [bucket * L + lane], ones_i32, mask=match)
        plsc.store_scatter(cbuf2, [lane * seg + cnt],
                           lax.bitcast_convert_type(uk, jnp.int32), mask=match)
        return cnt + jnp.where(match, 1, 0)

      b2, k3, _m2 = walk(k2)
      max2 = jnp.max(cnt2)
      b2_u = lax.convert_element_type(b2, jnp.uint32)

      clear_hist()

      @plsc.parallel_loop(0, max2, 1)
      def _(j):
        uk = lax.bitcast_convert_type(
            plsc.load_gather(cbuf2, [lane * seg + j]), jnp.uint32)
        valid = j < cnt2
        match = valid & (
            (jnp.uint32(0xFF) & lax.shift_right_logical(uk, jnp.uint32(8)))
            == b2_u)
        bucket = lax.convert_element_type(jnp.uint32(0xFF) & uk, jnp.int32)
        plsc.addupdate_scatter(hist, [bucket * L + lane], ones_i32, mask=match)

      b3, _k4, _m3 = walk(k3)

      thr_u = (lax.shift_left(b0_u, jnp.uint32(24))
               | lax.shift_left(b1_u, jnp.uint32(16))
               | lax.shift_left(lax.convert_element_type(b3, jnp.uint32),
                                jnp.uint32(0))
               | lax.shift_left(b2_u, jnp.uint32(8)))
      neg = thr_u < jnp.uint32(0x80000000)
      thr_bits = thr_u ^ jnp.where(neg, jnp.uint32(0xFFFFFFFF),
                                   jnp.uint32(0x80000000))
      thr_f = lax.bitcast_convert_type(thr_bits, jnp.float32)

      @plsc.parallel_loop(0, emb, L, unroll=UNR)
      def _(i):
        sl = pl.ds(i, L)
        xbuf[sl] = jnp.where(xbuf[sl] >= thr_f, jnp.float32(1.0),
                             jnp.float32(0.0))

      pltpu.sync_copy(xbuf, out_hbm.at[row])
      return 0

    lax.fori_loop(0, rows_per_w, do_row, 0)

  return pl.kernel(
      body,
      out_type=jax.ShapeDtypeStruct((batch, emb), jnp.float32),
      mesh=mesh,
      compiler_params=pltpu.CompilerParams(needs_layout_passes=False),
      scratch_types=[
          pltpu.VMEM((emb,), jnp.float32),
          pltpu.VMEM((emb,), jnp.int32),
          pltpu.VMEM((emb,), jnp.int32),
          pltpu.VMEM((NB * L,), jnp.int32),
      ],
  )


@jax.jit
def kernel(x):
  batch, emb = x.shape
  k = math.ceil(SPARSITY * emb)
  return _kwta_sc(batch, emb, k, 32)(x)


# speedup vs baseline: 1.0229x; 1.0229x over previous
import functools
import math

import jax
import jax.numpy as jnp
from jax import lax
from jax.experimental import pallas as pl
from jax.experimental.pallas import tpu as pltpu
from jax.experimental.pallas import tpu_sc as plsc

L = 16
NB = 256
SPARSITY = 0.05


def _kwta_sc(batch, emb, k, n_workers):
  rows_per_w = batch // n_workers
  seg = emb // L
  UNR = 8

  mesh = plsc.VectorSubcoreMesh(core_axis_name="c", subcore_axis_name="s")

  def body(x_hbm, out_hbm, xbuf, cbuf, cbuf2, hist):
    cid = lax.axis_index("c")
    sid = lax.axis_index("s")
    wid = sid * 2 + cid
    lane = lax.iota(jnp.int32, L)
    ones_i32 = jnp.ones((L,), jnp.int32)

    def ukey_of(raw_f32):
      raw = lax.bitcast_convert_type(raw_f32, jnp.uint32)
      neg = raw >= jnp.uint32(0x80000000)
      flip = jnp.where(neg, jnp.uint32(0xFFFFFFFF), jnp.uint32(0x80000000))
      return raw ^ flip

    def clear_hist():
      @plsc.parallel_loop(0, NB * L, L, unroll=UNR)
      def _(i):
        hist[pl.ds(i, L)] = jnp.zeros((L,), jnp.int32)

    def walk(k_rem):
      carry = jnp.int32(0)
      bucket_sel = jnp.int32(0)
      k_next = jnp.int32(0)
      m_sel = jnp.int32(0)
      for g in range(NB // L - 1, -1, -1):
        tg = jnp.zeros((L,), jnp.int32)
        for sub in range(L):
          gidx = g * NB + lane * L + sub
          tg = tg + plsc.load_gather(hist, [gidx])
        rc = plsc.cumsum(lax.rev(tg, (0,)))
        srev = lax.rev(rc, (0,))
        group_sum = jnp.max(rc)
        inc = srev + carry
        ge = inc >= k_rem
        cnt = jnp.max(plsc.all_reduce_population_count(ge))
        jstar = cnt - 1
        t_at = jnp.sum(jnp.where(lane == jstar, tg, 0))
        inc_at = jnp.sum(jnp.where(lane == jstar, inc, 0))
        cond = (carry < k_rem) & (carry + group_sum >= k_rem)
        bucket_sel = bucket_sel + jnp.where(cond, g * L + jstar, 0)
        k_next = k_next + jnp.where(cond, k_rem - (inc_at - t_at), 0)
        m_sel = m_sel + jnp.where(cond, t_at, 0)
        carry = carry + group_sum
      return bucket_sel, k_next, m_sel

    def do_row(r, carry_none):
      row = wid * rows_per_w + r
      pltpu.sync_copy(x_hbm.at[row], xbuf)

      clear_hist()

      @plsc.parallel_loop(0, emb, L, unroll=UNR)
      def _(i):
        uk = ukey_of(xbuf[pl.ds(i, L)])
        bucket = lax.convert_element_type(
            lax.shift_right_logical(uk, jnp.uint32(24)), jnp.int32)
        plsc.addupdate_scatter(hist, [bucket * L + lane], ones_i32)

      b0, k1, _m0 = walk(jnp.int32(k))

      clear_hist()
      b0_u = lax.convert_element_type(b0, jnp.uint32)

      @plsc.parallel_loop(0, emb, L, unroll=UNR, carry=jnp.zeros((L,), jnp.int32))
      def cnt1(i, cnt):
        uk = ukey_of(xbuf[pl.ds(i, L)])
        match = lax.shift_right_logical(uk, jnp.uint32(24)) == b0_u
        bucket = lax.convert_element_type(
            jnp.uint32(0xFF) & lax.shift_right_logical(uk, jnp.uint32(16)),
            jnp.int32)
        plsc.addupdate_scatter(hist, [bucket * L + lane], ones_i32, mask=match)
        plsc.store_scatter(cbuf, [lane * seg + cnt],
                           lax.bitcast_convert_type(uk, jnp.int32), mask=match)
        return cnt + jnp.where(match, 1, 0)

      b1, k2, _m1 = walk(k1)
      max1 = (jnp.max(cnt1) + (UNR - 1)) & ~(UNR - 1)
      b1_u = lax.convert_element_type(b1, jnp.uint32)

      clear_hist()

      @plsc.parallel_loop(0, max1, 1, unroll=UNR,
                          carry=jnp.zeros((L,), jnp.int32))
      def cnt2(j, cnt):
        uk = lax.bitcast_convert_type(
            plsc.load_gather(cbuf, [lane * seg + j]), jnp.uint32)
        valid = j < cnt1
        match = valid & (
            (jnp.uint32(0xFF) & lax.shift_right_logical(uk, jnp.uint32(16)))
            == b1_u)
        bucket = lax.convert_element_type(
            jnp.uint32(0xFF) & lax.shift_right_logical(uk, jnp.uint32(8)),
            jnp.int32)
        plsc.addupdate_scatter(hist, [bucket * L + lane], ones_i32, mask=match)
        plsc.store_scatter(cbuf2, [lane * seg + cnt],
                           lax.bitcast_convert_type(uk, jnp.int32), mask=match)
        return cnt + jnp.where(match, 1, 0)

      b2, k3, _m2 = walk(k2)
      max2 = (jnp.max(cnt2) + (UNR - 1)) & ~(UNR - 1)
      b2_u = lax.convert_element_type(b2, jnp.uint32)

      clear_hist()

      @plsc.parallel_loop(0, max2, 1, unroll=UNR)
      def _(j):
        uk = lax.bitcast_convert_type(
            plsc.load_gather(cbuf2, [lane * seg + j]), jnp.uint32)
        valid = j < cnt2
        match = valid & (
            (jnp.uint32(0xFF) & lax.shift_right_logical(uk, jnp.uint32(8)))
            == b2_u)
        bucket = lax.convert_element_type(jnp.uint32(0xFF) & uk, jnp.int32)
        plsc.addupdate_scatter(hist, [bucket * L + lane], ones_i32, mask=match)

      b3, _k4, _m3 = walk(k3)

      thr_u = (lax.shift_left(b0_u, jnp.uint32(24))
               | lax.shift_left(b1_u, jnp.uint32(16))
               | lax.shift_left(lax.convert_element_type(b3, jnp.uint32),
                                jnp.uint32(0))
               | lax.shift_left(b2_u, jnp.uint32(8)))
      neg = thr_u < jnp.uint32(0x80000000)
      thr_bits = thr_u ^ jnp.where(neg, jnp.uint32(0xFFFFFFFF),
                                   jnp.uint32(0x80000000))
      thr_f = lax.bitcast_convert_type(thr_bits, jnp.float32)

      @plsc.parallel_loop(0, emb, L, unroll=UNR)
      def _(i):
        sl = pl.ds(i, L)
        xbuf[sl] = jnp.where(xbuf[sl] >= thr_f, jnp.float32(1.0),
                             jnp.float32(0.0))

      pltpu.sync_copy(xbuf, out_hbm.at[row])
      return 0

    lax.fori_loop(0, rows_per_w, do_row, 0)

  return pl.kernel(
      body,
      out_type=jax.ShapeDtypeStruct((batch, emb), jnp.float32),
      mesh=mesh,
      compiler_params=pltpu.CompilerParams(needs_layout_passes=False),
      scratch_types=[
          pltpu.VMEM((emb,), jnp.float32),
          pltpu.VMEM((emb,), jnp.int32),
          pltpu.VMEM((emb,), jnp.int32),
          pltpu.VMEM((NB * L,), jnp.int32),
      ],
  )


@jax.jit
def kernel(x):
  batch, emb = x.shape
  k = math.ceil(SPARSITY * emb)
  return _kwta_sc(batch, emb, k, 32)(x)


# speedup vs baseline: 1.1747x; 1.1485x over previous
import functools
import math

import jax
import jax.numpy as jnp
from jax import lax
from jax.experimental import pallas as pl
from jax.experimental.pallas import tpu as pltpu
from jax.experimental.pallas import tpu_sc as plsc

L = 16
NB = 256
SPARSITY = 0.05


def _kwta_sc(batch, emb, k, n_workers):
  rows_per_w = batch // n_workers
  UNR = 8

  mesh = plsc.VectorSubcoreMesh(core_axis_name="c", subcore_axis_name="s")

  def body(x_hbm, out_hbm, xbuf, cbuf, cbuf2, hist):
    cid = lax.axis_index("c")
    sid = lax.axis_index("s")
    wid = sid * 2 + cid
    lane = lax.iota(jnp.int32, L)
    ones_i32 = jnp.ones((L,), jnp.int32)

    def ukey_of(raw_f32):
      raw = lax.bitcast_convert_type(raw_f32, jnp.uint32)
      neg = raw >= jnp.uint32(0x80000000)
      flip = jnp.where(neg, jnp.uint32(0xFFFFFFFF), jnp.uint32(0x80000000))
      return raw ^ flip

    def clear_hist():
      @plsc.parallel_loop(0, NB * L, L, unroll=UNR)
      def _(i):
        hist[pl.ds(i, L)] = jnp.zeros((L,), jnp.int32)

    def walk(k_rem):
      carry = jnp.int32(0)
      bucket_sel = jnp.int32(0)
      k_next = jnp.int32(0)
      m_sel = jnp.int32(0)
      for g in range(NB // L - 1, -1, -1):
        tg = jnp.zeros((L,), jnp.int32)
        for sub in range(L):
          gidx = g * NB + lane * L + sub
          tg = tg + plsc.load_gather(hist, [gidx])
        rc = plsc.cumsum(lax.rev(tg, (0,)))
        srev = lax.rev(rc, (0,))
        group_sum = jnp.max(rc)
        inc = srev + carry
        ge = inc >= k_rem
        cnt = jnp.max(plsc.all_reduce_population_count(ge))
        jstar = cnt - 1
        t_at = jnp.sum(jnp.where(lane == jstar, tg, 0))
        inc_at = jnp.sum(jnp.where(lane == jstar, inc, 0))
        cond = (carry < k_rem) & (carry + group_sum >= k_rem)
        bucket_sel = bucket_sel + jnp.where(cond, g * L + jstar, 0)
        k_next = k_next + jnp.where(cond, k_rem - (inc_at - t_at), 0)
        m_sel = m_sel + jnp.where(cond, t_at, 0)
        carry = carry + group_sum
      return bucket_sel, k_next, m_sel

    def do_row(r, carry_none):
      row = wid * rows_per_w + r
      pltpu.sync_copy(x_hbm.at[row], xbuf)

      clear_hist()

      @plsc.parallel_loop(0, emb, L, unroll=UNR)
      def _(i):
        uk = ukey_of(xbuf[pl.ds(i, L)])
        bucket = lax.convert_element_type(
            lax.shift_right_logical(uk, jnp.uint32(24)), jnp.int32)
        plsc.addupdate_scatter(hist, [bucket * L + lane], ones_i32)

      b0, k1, _m0 = walk(jnp.int32(k))

      clear_hist()
      b0_u = lax.convert_element_type(b0, jnp.uint32)

      @plsc.parallel_loop(0, emb, L, unroll=UNR, carry=jnp.zeros((L,), jnp.int32))
      def cnt1(i, cnt):
        uk = ukey_of(xbuf[pl.ds(i, L)])
        match = lax.shift_right_logical(uk, jnp.uint32(24)) == b0_u
        bucket = lax.convert_element_type(
            jnp.uint32(0xFF) & lax.shift_right_logical(uk, jnp.uint32(16)),
            jnp.int32)
        plsc.addupdate_scatter(hist, [bucket * L + lane], ones_i32, mask=match)
        plsc.store_scatter(cbuf, [cnt * L + lane],
                           lax.bitcast_convert_type(uk, jnp.int32), mask=match)
        return cnt + jnp.where(match, 1, 0)

      b1, k2, _m1 = walk(k1)
      max1 = ((jnp.max(cnt1) + (UNR - 1)) & ~(UNR - 1)) * L
      b1_u = lax.convert_element_type(b1, jnp.uint32)

      clear_hist()

      @plsc.parallel_loop(0, max1, L, unroll=UNR,
                          carry=jnp.zeros((L,), jnp.int32))
      def cnt2(i, cnt):
        uk = lax.bitcast_convert_type(cbuf[pl.ds(i, L)], jnp.uint32)
        valid = i < cnt1 * L
        match = valid & (
            (jnp.uint32(0xFF) & lax.shift_right_logical(uk, jnp.uint32(16)))
            == b1_u)
        bucket = lax.convert_element_type(
            jnp.uint32(0xFF) & lax.shift_right_logical(uk, jnp.uint32(8)),
            jnp.int32)
        plsc.addupdate_scatter(hist, [bucket * L + lane], ones_i32, mask=match)
        plsc.store_scatter(cbuf2, [cnt * L + lane],
                           lax.bitcast_convert_type(uk, jnp.int32), mask=match)
        return cnt + jnp.where(match, 1, 0)

      b2, k3, _m2 = walk(k2)
      max2 = ((jnp.max(cnt2) + (UNR - 1)) & ~(UNR - 1)) * L
      b2_u = lax.convert_element_type(b2, jnp.uint32)

      clear_hist()

      @plsc.parallel_loop(0, max2, L, unroll=UNR)
      def _(i):
        uk = lax.bitcast_convert_type(cbuf2[pl.ds(i, L)], jnp.uint32)
        valid = i < cnt2 * L
        match = valid & (
            (jnp.uint32(0xFF) & lax.shift_right_logical(uk, jnp.uint32(8)))
            == b2_u)
        bucket = lax.convert_element_type(jnp.uint32(0xFF) & uk, jnp.int32)
        plsc.addupdate_scatter(hist, [bucket * L + lane], ones_i32, mask=match)

      b3, _k4, _m3 = walk(k3)

      thr_u = (lax.shift_left(b0_u, jnp.uint32(24))
               | lax.shift_left(b1_u, jnp.uint32(16))
               | lax.shift_left(lax.convert_element_type(b3, jnp.uint32),
                                jnp.uint32(0))
               | lax.shift_left(b2_u, jnp.uint32(8)))
      neg = thr_u < jnp.uint32(0x80000000)
      thr_bits = thr_u ^ jnp.where(neg, jnp.uint32(0xFFFFFFFF),
                                   jnp.uint32(0x80000000))
      thr_f = lax.bitcast_convert_type(thr_bits, jnp.float32)

      @plsc.parallel_loop(0, emb, L, unroll=UNR)
      def _(i):
        sl = pl.ds(i, L)
        xbuf[sl] = jnp.where(xbuf[sl] >= thr_f, jnp.float32(1.0),
                             jnp.float32(0.0))

      pltpu.sync_copy(xbuf, out_hbm.at[row])
      return 0

    lax.fori_loop(0, rows_per_w, do_row, 0)

  return pl.kernel(
      body,
      out_type=jax.ShapeDtypeStruct((batch, emb), jnp.float32),
      mesh=mesh,
      compiler_params=pltpu.CompilerParams(needs_layout_passes=False),
      scratch_types=[
          pltpu.VMEM((emb,), jnp.float32),
          pltpu.VMEM((emb,), jnp.int32),
          pltpu.VMEM((emb,), jnp.int32),
          pltpu.VMEM((NB * L,), jnp.int32),
      ],
  )


@jax.jit
def kernel(x):
  batch, emb = x.shape
  k = math.ceil(SPARSITY * emb)
  return _kwta_sc(batch, emb, k, 32)(x)


# speedup vs baseline: 1.4919x; 1.2700x over previous
import functools
import math

import jax
import jax.numpy as jnp
from jax import lax
from jax.experimental import pallas as pl
from jax.experimental.pallas import tpu as pltpu
from jax.experimental.pallas import tpu_sc as plsc

L = 16
NB = 256
SPARSITY = 0.05


def _kwta_sc(batch, emb, k, n_workers):
  rows_per_w = batch // n_workers
  UNR = 8

  mesh = plsc.VectorSubcoreMesh(core_axis_name="c", subcore_axis_name="s")

  def body(x_hbm, out_hbm, xbuf, cbuf, cbuf2, hist):
    cid = lax.axis_index("c")
    sid = lax.axis_index("s")
    wid = sid * 2 + cid
    lane = lax.iota(jnp.int32, L)
    ones_i32 = jnp.ones((L,), jnp.int32)

    def ukey_of(raw_f32):
      raw = lax.bitcast_convert_type(raw_f32, jnp.uint32)
      neg = raw >= jnp.uint32(0x80000000)
      flip = jnp.where(neg, jnp.uint32(0xFFFFFFFF), jnp.uint32(0x80000000))
      return raw ^ flip

    def clear_hist():
      @plsc.parallel_loop(0, NB * L, L, unroll=UNR)
      def _(i):
        hist[pl.ds(i, L)] = jnp.zeros((L,), jnp.int32)

    def walk(k_rem):
      carry = jnp.int32(0)
      g_sel = jnp.int32(0)
      c_above = jnp.int32(0)
      for g in range(NB // L - 1, -1, -1):
        s = jnp.zeros((L,), jnp.int32)
        for j in range(L):
          s = s + hist[pl.ds((g * L + j) * L, L)]
        group_sum = jnp.max(plsc.cumsum(s))
        cond = (carry < k_rem) & (carry + group_sum >= k_rem)
        g_sel = g_sel + jnp.where(cond, g, 0)
        c_above = c_above + jnp.where(cond, carry, 0)
        carry = carry + group_sum
      base = g_sel * (L * L)
      tg = jnp.zeros((L,), jnp.int32)
      for j in range(L):
        v = hist[pl.ds(base + j * L, L)]
        tj = jnp.max(plsc.cumsum(v))
        tg = tg + jnp.where(lane == j, tj, 0)
      rc = plsc.cumsum(lax.rev(tg, (0,)))
      srev = lax.rev(rc, (0,))
      inc = srev + c_above
      ge = inc >= k_rem
      cnt = jnp.max(plsc.all_reduce_population_count(ge))
      jstar = cnt - 1
      t_at = jnp.sum(jnp.where(lane == jstar, tg, 0))
      inc_at = jnp.sum(jnp.where(lane == jstar, inc, 0))
      bucket_sel = g_sel * L + jstar
      k_next = k_rem - (inc_at - t_at)
      m_sel = t_at
      return bucket_sel, k_next, m_sel

    def do_row(r, carry_none):
      row = wid * rows_per_w + r
      pltpu.sync_copy(x_hbm.at[row], xbuf)

      clear_hist()

      @plsc.parallel_loop(0, emb, L, unroll=UNR)
      def _(i):
        uk = ukey_of(xbuf[pl.ds(i, L)])
        bucket = lax.convert_element_type(
            lax.shift_right_logical(uk, jnp.uint32(24)), jnp.int32)
        plsc.addupdate_scatter(hist, [bucket * L + lane], ones_i32)

      b0, k1, _m0 = walk(jnp.int32(k))

      clear_hist()
      b0_u = lax.convert_element_type(b0, jnp.uint32)

      @plsc.parallel_loop(0, emb, L, unroll=UNR, carry=jnp.zeros((L,), jnp.int32))
      def cnt1(i, cnt):
        uk = ukey_of(xbuf[pl.ds(i, L)])
        match = lax.shift_right_logical(uk, jnp.uint32(24)) == b0_u
        bucket = lax.convert_element_type(
            jnp.uint32(0xFF) & lax.shift_right_logical(uk, jnp.uint32(16)),
            jnp.int32)
        plsc.addupdate_scatter(hist, [bucket * L + lane], ones_i32, mask=match)
        plsc.store_scatter(cbuf, [cnt * L + lane],
                           lax.bitcast_convert_type(uk, jnp.int32), mask=match)
        return cnt + jnp.where(match, 1, 0)

      b1, k2, _m1 = walk(k1)
      max1 = ((jnp.max(cnt1) + (UNR - 1)) & ~(UNR - 1)) * L
      b1_u = lax.convert_element_type(b1, jnp.uint32)

      clear_hist()

      @plsc.parallel_loop(0, max1, L, unroll=UNR,
                          carry=jnp.zeros((L,), jnp.int32))
      def cnt2(i, cnt):
        uk = lax.bitcast_convert_type(cbuf[pl.ds(i, L)], jnp.uint32)
        valid = i < cnt1 * L
        match = valid & (
            (jnp.uint32(0xFF) & lax.shift_right_logical(uk, jnp.uint32(16)))
            == b1_u)
        bucket = lax.convert_element_type(
            jnp.uint32(0xFF) & lax.shift_right_logical(uk, jnp.uint32(8)),
            jnp.int32)
        plsc.addupdate_scatter(hist, [bucket * L + lane], ones_i32, mask=match)
        plsc.store_scatter(cbuf2, [cnt * L + lane],
                           lax.bitcast_convert_type(uk, jnp.int32), mask=match)
        return cnt + jnp.where(match, 1, 0)

      b2, k3, _m2 = walk(k2)
      max2 = ((jnp.max(cnt2) + (UNR - 1)) & ~(UNR - 1)) * L
      b2_u = lax.convert_element_type(b2, jnp.uint32)

      clear_hist()

      @plsc.parallel_loop(0, max2, L, unroll=UNR)
      def _(i):
        uk = lax.bitcast_convert_type(cbuf2[pl.ds(i, L)], jnp.uint32)
        valid = i < cnt2 * L
        match = valid & (
            (jnp.uint32(0xFF) & lax.shift_right_logical(uk, jnp.uint32(8)))
            == b2_u)
        bucket = lax.convert_element_type(jnp.uint32(0xFF) & uk, jnp.int32)
        plsc.addupdate_scatter(hist, [bucket * L + lane], ones_i32, mask=match)

      b3, _k4, _m3 = walk(k3)

      thr_u = (lax.shift_left(b0_u, jnp.uint32(24))
               | lax.shift_left(b1_u, jnp.uint32(16))
               | lax.shift_left(lax.convert_element_type(b3, jnp.uint32),
                                jnp.uint32(0))
               | lax.shift_left(b2_u, jnp.uint32(8)))
      neg = thr_u < jnp.uint32(0x80000000)
      thr_bits = thr_u ^ jnp.where(neg, jnp.uint32(0xFFFFFFFF),
                                   jnp.uint32(0x80000000))
      thr_f = lax.bitcast_convert_type(thr_bits, jnp.float32)

      @plsc.parallel_loop(0, emb, L, unroll=UNR)
      def _(i):
        sl = pl.ds(i, L)
        xbuf[sl] = jnp.where(xbuf[sl] >= thr_f, jnp.float32(1.0),
                             jnp.float32(0.0))

      pltpu.sync_copy(xbuf, out_hbm.at[row])
      return 0

    lax.fori_loop(0, rows_per_w, do_row, 0)

  return pl.kernel(
      body,
      out_type=jax.ShapeDtypeStruct((batch, emb), jnp.float32),
      mesh=mesh,
      compiler_params=pltpu.CompilerParams(needs_layout_passes=False),
      scratch_types=[
          pltpu.VMEM((emb,), jnp.float32),
          pltpu.VMEM((emb,), jnp.int32),
          pltpu.VMEM((emb,), jnp.int32),
          pltpu.VMEM((NB * L,), jnp.int32),
      ],
  )


@jax.jit
def kernel(x):
  batch, emb = x.shape
  k = math.ceil(SPARSITY * emb)
  return _kwta_sc(batch, emb, k, 32)(x)


# speedup vs baseline: 1.6671x; 1.1174x over previous
import functools
import math

import jax
import jax.numpy as jnp
from jax import lax
from jax.experimental import pallas as pl
from jax.experimental.pallas import tpu as pltpu
from jax.experimental.pallas import tpu_sc as plsc

L = 16
NB = 256
SPARSITY = 0.05


def _kwta_sc(batch, emb, k, n_workers):
  rows_per_w = batch // n_workers
  UNR = 8

  mesh = plsc.VectorSubcoreMesh(core_axis_name="c", subcore_axis_name="s")

  def body(x_hbm, out_hbm, buf0, buf1, buf2, hist, sem0, sem1, sem2, semo):
    cid = lax.axis_index("c")
    sid = lax.axis_index("s")
    wid = sid * 2 + cid
    lane = lax.iota(jnp.int32, L)
    ones_i32 = jnp.ones((L,), jnp.int32)

    def ukey_of(raw_f32):
      raw = lax.bitcast_convert_type(raw_f32, jnp.uint32)
      neg = raw >= jnp.uint32(0x80000000)
      flip = jnp.where(neg, jnp.uint32(0xFFFFFFFF), jnp.uint32(0x80000000))
      return raw ^ flip

    def clear_hist():
      @plsc.parallel_loop(0, NB * L, L, unroll=UNR)
      def _(i):
        hist[pl.ds(i, L)] = jnp.zeros((L,), jnp.int32)

    def walk(k_rem):
      def g_body(gi, st):
        carry, g_sel, c_above = st
        g = (NB // L - 1) - gi
        s = jnp.zeros((L,), jnp.int32)
        for j in range(L):
          s = s + hist[pl.ds(g * (L * L) + j * L, L)]
        group_sum = jnp.max(plsc.cumsum(s))
        cond = (carry < k_rem) & (carry + group_sum >= k_rem)
        g_sel = g_sel + jnp.where(cond, g, 0)
        c_above = c_above + jnp.where(cond, carry, 0)
        return carry + group_sum, g_sel, c_above

      _, g_sel, c_above = lax.fori_loop(
          0, NB // L, g_body, (jnp.int32(0), jnp.int32(0), jnp.int32(0)))
      base = g_sel * (L * L)

      def j_body(j, tg):
        v = hist[pl.ds(base + j * L, L)]
        tj = jnp.max(plsc.cumsum(v))
        return tg + jnp.where(lane == j, tj, 0)

      tg = lax.fori_loop(0, L, j_body, jnp.zeros((L,), jnp.int32))
      rc = plsc.cumsum(lax.rev(tg, (0,)))
      srev = lax.rev(rc, (0,))
      inc = srev + c_above
      ge = inc >= k_rem
      cnt = jnp.max(plsc.all_reduce_population_count(ge))
      jstar = cnt - 1
      t_at = jnp.sum(jnp.where(lane == jstar, tg, 0))
      inc_at = jnp.sum(jnp.where(lane == jstar, inc, 0))
      bucket_sel = g_sel * L + jstar
      k_next = k_rem - (inc_at - t_at)
      m_sel = t_at
      return bucket_sel, k_next, m_sel

    def thresh_of(xbuf, cbuf, drain):
      clear_hist()

      @plsc.parallel_loop(0, emb, L, unroll=UNR)
      def _(i):
        uk = ukey_of(xbuf[pl.ds(i, L)])
        bucket = lax.convert_element_type(
            lax.shift_right_logical(uk, jnp.uint32(24)), jnp.int32)
        plsc.addupdate_scatter(hist, [bucket * L + lane], ones_i32)

      b0, k1, _m0 = walk(jnp.int32(k))

      drain()

      clear_hist()
      b0_u = lax.convert_element_type(b0, jnp.uint32)

      @plsc.parallel_loop(0, emb, L, unroll=UNR, carry=jnp.zeros((L,), jnp.int32))
      def cnt1(i, cnt):
        uk = ukey_of(xbuf[pl.ds(i, L)])
        match = lax.shift_right_logical(uk, jnp.uint32(24)) == b0_u
        bucket = lax.convert_element_type(
            jnp.uint32(0xFF) & lax.shift_right_logical(uk, jnp.uint32(16)),
            jnp.int32)
        plsc.addupdate_scatter(hist, [bucket * L + lane], ones_i32, mask=match)
        plsc.store_scatter(cbuf, [cnt * L + lane],
                           lax.bitcast_convert_type(uk, jnp.float32),
                           mask=match)
        return cnt + jnp.where(match, 1, 0)

      b1, k2, _m1 = walk(k1)
      max1 = ((jnp.max(cnt1) + (UNR - 1)) & ~(UNR - 1)) * L
      b1_u = lax.convert_element_type(b1, jnp.uint32)

      clear_hist()

      @plsc.parallel_loop(0, max1, L, unroll=UNR,
                          carry=jnp.zeros((L,), jnp.int32))
      def cnt2(i, cnt):
        uk = lax.bitcast_convert_type(cbuf[pl.ds(i, L)], jnp.uint32)
        valid = i < cnt1 * L
        match = valid & (
            (jnp.uint32(0xFF) & lax.shift_right_logical(uk, jnp.uint32(16)))
            == b1_u)
        bucket = lax.convert_element_type(
            jnp.uint32(0xFF) & lax.shift_right_logical(uk, jnp.uint32(8)),
            jnp.int32)
        plsc.addupdate_scatter(hist, [bucket * L + lane], ones_i32, mask=match)
        plsc.store_scatter(cbuf, [cnt * L + lane],
                           lax.bitcast_convert_type(uk, jnp.float32),
                           mask=match)
        return cnt + jnp.where(match, 1, 0)

      b2, k3, _m2 = walk(k2)
      max2 = ((jnp.max(cnt2) + (UNR - 1)) & ~(UNR - 1)) * L
      b2_u = lax.convert_element_type(b2, jnp.uint32)

      clear_hist()

      @plsc.parallel_loop(0, max2, L, unroll=UNR)
      def _(i):
        uk = lax.bitcast_convert_type(cbuf[pl.ds(i, L)], jnp.uint32)
        valid = i < cnt2 * L
        match = valid & (
            (jnp.uint32(0xFF) & lax.shift_right_logical(uk, jnp.uint32(8)))
            == b2_u)
        bucket = lax.convert_element_type(jnp.uint32(0xFF) & uk, jnp.int32)
        plsc.addupdate_scatter(hist, [bucket * L + lane], ones_i32, mask=match)

      b3, _k4, _m3 = walk(k3)

      thr_u = (lax.shift_left(b0_u, jnp.uint32(24))
               | lax.shift_left(b1_u, jnp.uint32(16))
               | lax.shift_left(lax.convert_element_type(b3, jnp.uint32),
                                jnp.uint32(0))
               | lax.shift_left(b2_u, jnp.uint32(8)))
      neg = thr_u < jnp.uint32(0x80000000)
      thr_bits = thr_u ^ jnp.where(neg, jnp.uint32(0xFFFFFFFF),
                                   jnp.uint32(0x80000000))
      return lax.bitcast_convert_type(thr_bits, jnp.float32)

    bufs = (buf0, buf1, buf2)
    isems = (sem0, sem1, sem2)
    base_row = wid * rows_per_w
    in_d = [None] * rows_per_w
    out_d = [None] * rows_per_w
    for r in range(min(2, rows_per_w)):
      in_d[r] = pltpu.async_copy(x_hbm.at[base_row + r], bufs[r % 3],
                                 isems[r % 3])
    for r in range(rows_per_w):
      xbuf = bufs[r % 3]
      cbuf = bufs[(r + 2) % 3]
      in_d[r].wait()
      drain = out_d[r - 1].wait if r >= 1 else (lambda: None)
      out_d[r - 1] = None
      thr_f = thresh_of(xbuf, cbuf, drain)

      @plsc.parallel_loop(0, emb, L, unroll=UNR)
      def _(i):
        sl = pl.ds(i, L)
        xbuf[sl] = jnp.where(xbuf[sl] >= thr_f, jnp.float32(1.0),
                             jnp.float32(0.0))

      out_d[r] = pltpu.async_copy(xbuf, out_hbm.at[base_row + r], semo)
      if r + 2 < rows_per_w:
        in_d[r + 2] = pltpu.async_copy(x_hbm.at[base_row + r + 2], cbuf,
                                       isems[(r + 2) % 3])
    out_d[rows_per_w - 1].wait()

  return pl.kernel(
      body,
      out_type=jax.ShapeDtypeStruct((batch, emb), jnp.float32),
      mesh=mesh,
      compiler_params=pltpu.CompilerParams(needs_layout_passes=False),
      scratch_types=[
          pltpu.VMEM((emb,), jnp.float32),
          pltpu.VMEM((emb,), jnp.float32),
          pltpu.VMEM((emb,), jnp.float32),
          pltpu.VMEM((NB * L,), jnp.int32),
          pltpu.SemaphoreType.DMA,
          pltpu.SemaphoreType.DMA,
          pltpu.SemaphoreType.DMA,
          pltpu.SemaphoreType.DMA,
      ],
  )


@jax.jit
def kernel(x):
  batch, emb = x.shape
  k = math.ceil(SPARSITY * emb)
  return _kwta_sc(batch, emb, k, 32)(x)


# speedup vs baseline: 1.7254x; 1.0349x over previous
import functools
import math

import jax
import jax.numpy as jnp
from jax import lax
from jax.experimental import pallas as pl
from jax.experimental.pallas import tpu as pltpu
from jax.experimental.pallas import tpu_sc as plsc

L = 16
NB = 256
SPARSITY = 0.05


def _kwta_sc(batch, emb, k, n_workers):
  rows_per_w = batch // n_workers
  UNR = 8

  mesh = plsc.VectorSubcoreMesh(core_axis_name="c", subcore_axis_name="s")

  def body(x_hbm, out_hbm, buf0, buf1, buf2, hist, sem0, sem1, sem2, semo):
    cid = lax.axis_index("c")
    sid = lax.axis_index("s")
    wid = sid * 2 + cid
    lane = lax.iota(jnp.int32, L)
    ones_i32 = jnp.ones((L,), jnp.int32)

    def ukey_of(raw_f32):
      raw = lax.bitcast_convert_type(raw_f32, jnp.uint32)
      neg = raw >= jnp.uint32(0x80000000)
      flip = jnp.where(neg, jnp.uint32(0xFFFFFFFF), jnp.uint32(0x80000000))
      return raw ^ flip

    def clear_hist():
      @plsc.parallel_loop(0, NB * L, L, unroll=UNR)
      def _(i):
        hist[pl.ds(i, L)] = jnp.zeros((L,), jnp.int32)

    def walk(k_rem):
      def g_body(gi, st):
        carry, g_sel, c_above = st
        g = (NB // L - 1) - gi
        s = jnp.zeros((L,), jnp.int32)
        for j in range(L):
          s = s + hist[pl.ds(g * (L * L) + j * L, L)]
        group_sum = jnp.max(plsc.cumsum(s))
        cond = (carry < k_rem) & (carry + group_sum >= k_rem)
        g_sel = g_sel + jnp.where(cond, g, 0)
        c_above = c_above + jnp.where(cond, carry, 0)
        return carry + group_sum, g_sel, c_above

      _, g_sel, c_above = lax.fori_loop(
          0, NB // L, g_body, (jnp.int32(0), jnp.int32(0), jnp.int32(0)))
      base = g_sel * (L * L)

      def j_body(j, tg):
        v = hist[pl.ds(base + j * L, L)]
        tj = jnp.max(plsc.cumsum(v))
        return tg + jnp.where(lane == j, tj, 0)

      tg = lax.fori_loop(0, L, j_body, jnp.zeros((L,), jnp.int32))
      rc = plsc.cumsum(lax.rev(tg, (0,)))
      srev = lax.rev(rc, (0,))
      inc = srev + c_above
      ge = inc >= k_rem
      cnt = jnp.max(plsc.all_reduce_population_count(ge))
      jstar = cnt - 1
      t_at = jnp.sum(jnp.where(lane == jstar, tg, 0))
      inc_at = jnp.sum(jnp.where(lane == jstar, inc, 0))
      bucket_sel = g_sel * L + jstar
      k_next = k_rem - (inc_at - t_at)
      m_sel = t_at
      return bucket_sel, k_next, m_sel

    def thresh_of(xbuf, cbuf, drain):
      clear_hist()

      @plsc.parallel_loop(0, emb, L, unroll=UNR)
      def _(i):
        uk = ukey_of(xbuf[pl.ds(i, L)])
        bucket = lax.convert_element_type(
            lax.shift_right_logical(uk, jnp.uint32(24)), jnp.int32)
        plsc.addupdate_scatter(hist, [bucket * L + lane], ones_i32)

      b0, k1, _m0 = walk(jnp.int32(k))

      drain()

      b0_u = lax.convert_element_type(b0, jnp.uint32)

      @plsc.parallel_loop(0, emb, L, unroll=UNR, carry=jnp.zeros((L,), jnp.int32))
      def cnt1(i, cnt):
        uk = ukey_of(xbuf[pl.ds(i, L)])
        match = lax.shift_right_logical(uk, jnp.uint32(24)) == b0_u
        plsc.store_scatter(cbuf, [cnt * L + lane],
                           lax.bitcast_convert_type(uk, jnp.float32),
                           mask=match)
        return cnt + jnp.where(match, 1, 0)

      clear_hist()
      max1 = ((jnp.max(cnt1) + (UNR - 1)) & ~(UNR - 1)) * L

      @plsc.parallel_loop(0, max1, L, unroll=UNR)
      def _(i):
        uk = lax.bitcast_convert_type(cbuf[pl.ds(i, L)], jnp.uint32)
        valid = i < cnt1 * L
        bucket = lax.convert_element_type(
            jnp.uint32(0xFF) & lax.shift_right_logical(uk, jnp.uint32(16)),
            jnp.int32)
        plsc.addupdate_scatter(hist, [bucket * L + lane], ones_i32, mask=valid)

      b1, k2, _m1 = walk(k1)
      b1_u = lax.convert_element_type(b1, jnp.uint32)

      @plsc.parallel_loop(0, max1, L, unroll=UNR,
                          carry=jnp.zeros((L,), jnp.int32))
      def cnt2(i, cnt):
        uk = lax.bitcast_convert_type(cbuf[pl.ds(i, L)], jnp.uint32)
        valid = i < cnt1 * L
        match = valid & (
            (jnp.uint32(0xFF) & lax.shift_right_logical(uk, jnp.uint32(16)))
            == b1_u)
        plsc.store_scatter(cbuf, [cnt * L + lane],
                           lax.bitcast_convert_type(uk, jnp.float32),
                           mask=match)
        return cnt + jnp.where(match, 1, 0)

      clear_hist()
      max2 = ((jnp.max(cnt2) + (UNR - 1)) & ~(UNR - 1)) * L

      @plsc.parallel_loop(0, max2, L, unroll=UNR)
      def _(i):
        uk = lax.bitcast_convert_type(cbuf[pl.ds(i, L)], jnp.uint32)
        valid = i < cnt2 * L
        bucket = lax.convert_element_type(
            jnp.uint32(0xFF) & lax.shift_right_logical(uk, jnp.uint32(8)),
            jnp.int32)
        plsc.addupdate_scatter(hist, [bucket * L + lane], ones_i32, mask=valid)

      b2, k3, _m2 = walk(k2)
      b2_u = lax.convert_element_type(b2, jnp.uint32)

      @plsc.parallel_loop(0, max2, L, unroll=UNR,
                          carry=jnp.zeros((L,), jnp.int32))
      def cnt3(i, cnt):
        uk = lax.bitcast_convert_type(cbuf[pl.ds(i, L)], jnp.uint32)
        valid = i < cnt2 * L
        match = valid & (
            (jnp.uint32(0xFF) & lax.shift_right_logical(uk, jnp.uint32(8)))
            == b2_u)
        plsc.store_scatter(cbuf, [cnt * L + lane],
                           lax.bitcast_convert_type(uk, jnp.float32),
                           mask=match)
        return cnt + jnp.where(match, 1, 0)

      clear_hist()
      max3 = ((jnp.max(cnt3) + (UNR - 1)) & ~(UNR - 1)) * L

      @plsc.parallel_loop(0, max3, L, unroll=UNR)
      def _(i):
        uk = lax.bitcast_convert_type(cbuf[pl.ds(i, L)], jnp.uint32)
        valid = i < cnt3 * L
        bucket = lax.convert_element_type(jnp.uint32(0xFF) & uk, jnp.int32)
        plsc.addupdate_scatter(hist, [bucket * L + lane], ones_i32, mask=valid)

      b3, _k4, _m3 = walk(k3)

      thr_u = (lax.shift_left(b0_u, jnp.uint32(24))
               | lax.shift_left(b1_u, jnp.uint32(16))
               | lax.shift_left(lax.convert_element_type(b3, jnp.uint32),
                                jnp.uint32(0))
               | lax.shift_left(b2_u, jnp.uint32(8)))
      neg = thr_u < jnp.uint32(0x80000000)
      thr_bits = thr_u ^ jnp.where(neg, jnp.uint32(0xFFFFFFFF),
                                   jnp.uint32(0x80000000))
      return lax.bitcast_convert_type(thr_bits, jnp.float32)

    bufs = (buf0, buf1, buf2)
    isems = (sem0, sem1, sem2)
    base_row = wid * rows_per_w
    in_d = [None] * rows_per_w
    out_d = [None] * rows_per_w
    for r in range(min(2, rows_per_w)):
      in_d[r] = pltpu.async_copy(x_hbm.at[base_row + r], bufs[r % 3],
                                 isems[r % 3])
    for r in range(rows_per_w):
      xbuf = bufs[r % 3]
      cbuf = bufs[(r + 2) % 3]
      in_d[r].wait()
      drain = out_d[r - 1].wait if r >= 1 else (lambda: None)
      out_d[r - 1] = None
      thr_f = thresh_of(xbuf, cbuf, drain)

      @plsc.parallel_loop(0, emb, L, unroll=UNR)
      def _(i):
        sl = pl.ds(i, L)
        xbuf[sl] = jnp.where(xbuf[sl] >= thr_f, jnp.float32(1.0),
                             jnp.float32(0.0))

      out_d[r] = pltpu.async_copy(xbuf, out_hbm.at[base_row + r], semo)
      if r + 2 < rows_per_w:
        in_d[r + 2] = pltpu.async_copy(x_hbm.at[base_row + r + 2], cbuf,
                                       isems[(r + 2) % 3])
    out_d[rows_per_w - 1].wait()

  return pl.kernel(
      body,
      out_type=jax.ShapeDtypeStruct((batch, emb), jnp.float32),
      mesh=mesh,
      compiler_params=pltpu.CompilerParams(needs_layout_passes=False),
      scratch_types=[
          pltpu.VMEM((emb,), jnp.float32),
          pltpu.VMEM((emb,), jnp.float32),
          pltpu.VMEM((emb,), jnp.float32),
          pltpu.VMEM((NB * L,), jnp.int32),
          pltpu.SemaphoreType.DMA,
          pltpu.SemaphoreType.DMA,
          pltpu.SemaphoreType.DMA,
          pltpu.SemaphoreType.DMA,
      ],
  )


@jax.jit
def kernel(x):
  batch, emb = x.shape
  k = math.ceil(SPARSITY * emb)
  return _kwta_sc(batch, emb, k, 32)(x)


# speedup vs baseline: 1.7721x; 1.0271x over previous
import functools
import math

import jax
import jax.numpy as jnp
from jax import lax
from jax.experimental import pallas as pl
from jax.experimental.pallas import tpu as pltpu
from jax.experimental.pallas import tpu_sc as plsc

L = 16
NB = 256
SPARSITY = 0.05


def _kwta_sc(batch, emb, k, n_workers):
  rows_per_w = batch // n_workers
  UNR = 8

  mesh = plsc.VectorSubcoreMesh(core_axis_name="c", subcore_axis_name="s")

  def body(x_hbm, out_hbm, buf0, buf1, buf2, hist, sem0, sem1, sem2, semo):
    cid = lax.axis_index("c")
    sid = lax.axis_index("s")
    wid = sid * 2 + cid
    lane = lax.iota(jnp.int32, L)
    ones_i32 = jnp.ones((L,), jnp.int32)

    def ukey_of(raw_f32):
      raw = lax.bitcast_convert_type(raw_f32, jnp.uint32)
      neg = raw >= jnp.uint32(0x80000000)
      flip = jnp.where(neg, jnp.uint32(0xFFFFFFFF), jnp.uint32(0x80000000))
      return raw ^ flip

    def clear_hist():
      @plsc.parallel_loop(0, NB * L, L, unroll=UNR)
      def _(i):
        hist[pl.ds(i, L)] = jnp.zeros((L,), jnp.int32)

    def walk(k_rem):
      def g_body(gi, st):
        carry, g_sel, c_above = st
        g = (NB // L - 1) - gi
        s = jnp.zeros((L,), jnp.int32)
        for j in range(L):
          s = s + hist[pl.ds(g * (L * L) + j * L, L)]
        group_sum = jnp.max(plsc.cumsum(s))
        cond = (carry < k_rem) & (carry + group_sum >= k_rem)
        g_sel = g_sel + jnp.where(cond, g, 0)
        c_above = c_above + jnp.where(cond, carry, 0)
        return carry + group_sum, g_sel, c_above

      _, g_sel, c_above = lax.fori_loop(
          0, NB // L, g_body, (jnp.int32(0), jnp.int32(0), jnp.int32(0)))
      base = g_sel * (L * L)

      def j_body(j, tg):
        v = hist[pl.ds(base + j * L, L)]
        tj = jnp.max(plsc.cumsum(v))
        return tg + jnp.where(lane == j, tj, 0)

      tg = lax.fori_loop(0, L, j_body, jnp.zeros((L,), jnp.int32))
      rc = plsc.cumsum(lax.rev(tg, (0,)))
      srev = lax.rev(rc, (0,))
      inc = srev + c_above
      ge = inc >= k_rem
      cnt = jnp.max(plsc.all_reduce_population_count(ge))
      jstar = cnt - 1
      t_at = jnp.sum(jnp.where(lane == jstar, tg, 0))
      inc_at = jnp.sum(jnp.where(lane == jstar, inc, 0))
      bucket_sel = g_sel * L + jstar
      k_next = k_rem - (inc_at - t_at)
      m_sel = t_at
      return bucket_sel, k_next, m_sel

    def thresh_of(xbuf, cbuf, drain):
      clear_hist()

      @plsc.parallel_loop(0, emb, L, unroll=UNR)
      def _(i):
        uk = ukey_of(xbuf[pl.ds(i, L)])
        bucket = lax.convert_element_type(
            lax.shift_right_logical(uk, jnp.uint32(24)), jnp.int32)
        plsc.addupdate_scatter(hist, [bucket * L + lane], ones_i32)

      b0, k1, _m0 = walk(jnp.int32(k))

      drain()

      b0_u = lax.convert_element_type(b0, jnp.uint32)
      b0_raw = jnp.where(b0_u >= jnp.uint32(128), b0_u - jnp.uint32(128),
                         jnp.uint32(255) - b0_u)

      @plsc.parallel_loop(0, emb, L, unroll=UNR, carry=jnp.zeros((L,), jnp.int32))
      def cnt1(i, cnt):
        xv = xbuf[pl.ds(i, L)]
        raw = lax.bitcast_convert_type(xv, jnp.uint32)
        match = lax.shift_right_logical(raw, jnp.uint32(24)) == b0_raw
        plsc.store_scatter(cbuf, [cnt * L + lane], xv, mask=match)
        return cnt + jnp.where(match, 1, 0)

      clear_hist()
      max1 = ((jnp.max(cnt1) + (UNR - 1)) & ~(UNR - 1)) * L

      @plsc.parallel_loop(0, max1, L, unroll=UNR)
      def _(i):
        uk = ukey_of(cbuf[pl.ds(i, L)])
        valid = i < cnt1 * L
        bucket = lax.convert_element_type(
            jnp.uint32(0xFF) & lax.shift_right_logical(uk, jnp.uint32(16)),
            jnp.int32)
        plsc.addupdate_scatter(hist, [bucket * L + lane], ones_i32, mask=valid)

      b1, k2, _m1 = walk(k1)
      b1_u = lax.convert_element_type(b1, jnp.uint32)

      @plsc.parallel_loop(0, max1, L, unroll=UNR,
                          carry=jnp.zeros((L,), jnp.int32))
      def cnt2(i, cnt):
        xv = cbuf[pl.ds(i, L)]
        uk = ukey_of(xv)
        valid = i < cnt1 * L
        match = valid & (
            (jnp.uint32(0xFF) & lax.shift_right_logical(uk, jnp.uint32(16)))
            == b1_u)
        plsc.store_scatter(cbuf, [cnt * L + lane], xv, mask=match)
        return cnt + jnp.where(match, 1, 0)

      clear_hist()
      max2 = ((jnp.max(cnt2) + (UNR - 1)) & ~(UNR - 1)) * L

      @plsc.parallel_loop(0, max2, L, unroll=UNR)
      def _(i):
        uk = ukey_of(cbuf[pl.ds(i, L)])
        valid = i < cnt2 * L
        bucket = lax.convert_element_type(
            jnp.uint32(0xFF) & lax.shift_right_logical(uk, jnp.uint32(8)),
            jnp.int32)
        plsc.addupdate_scatter(hist, [bucket * L + lane], ones_i32, mask=valid)

      b2, k3, _m2 = walk(k2)
      b2_u = lax.convert_element_type(b2, jnp.uint32)

      @plsc.parallel_loop(0, max2, L, unroll=UNR,
                          carry=jnp.zeros((L,), jnp.int32))
      def cnt3(i, cnt):
        xv = cbuf[pl.ds(i, L)]
        uk = ukey_of(xv)
        valid = i < cnt2 * L
        match = valid & (
            (jnp.uint32(0xFF) & lax.shift_right_logical(uk, jnp.uint32(8)))
            == b2_u)
        plsc.store_scatter(cbuf, [cnt * L + lane], xv, mask=match)
        return cnt + jnp.where(match, 1, 0)

      clear_hist()
      max3 = ((jnp.max(cnt3) + (UNR - 1)) & ~(UNR - 1)) * L

      @plsc.parallel_loop(0, max3, L, unroll=UNR)
      def _(i):
        uk = ukey_of(cbuf[pl.ds(i, L)])
        valid = i < cnt3 * L
        bucket = lax.convert_element_type(jnp.uint32(0xFF) & uk, jnp.int32)
        plsc.addupdate_scatter(hist, [bucket * L + lane], ones_i32, mask=valid)

      b3, _k4, _m3 = walk(k3)

      thr_u = (lax.shift_left(b0_u, jnp.uint32(24))
               | lax.shift_left(b1_u, jnp.uint32(16))
               | lax.shift_left(lax.convert_element_type(b3, jnp.uint32),
                                jnp.uint32(0))
               | lax.shift_left(b2_u, jnp.uint32(8)))
      neg = thr_u < jnp.uint32(0x80000000)
      thr_bits = thr_u ^ jnp.where(neg, jnp.uint32(0xFFFFFFFF),
                                   jnp.uint32(0x80000000))
      return lax.bitcast_convert_type(thr_bits, jnp.float32)

    bufs = (buf0, buf1, buf2)
    isems = (sem0, sem1, sem2)
    base_row = wid * rows_per_w
    in_d = [None] * rows_per_w
    out_d = [None] * rows_per_w
    for r in range(min(2, rows_per_w)):
      in_d[r] = pltpu.async_copy(x_hbm.at[base_row + r], bufs[r % 3],
                                 isems[r % 3])
    for r in range(rows_per_w):
      xbuf = bufs[r % 3]
      cbuf = bufs[(r + 2) % 3]
      in_d[r].wait()
      drain = out_d[r - 1].wait if r >= 1 else (lambda: None)
      out_d[r - 1] = None
      thr_f = thresh_of(xbuf, cbuf, drain)

      @plsc.parallel_loop(0, emb, L, unroll=UNR)
      def _(i):
        sl = pl.ds(i, L)
        xbuf[sl] = jnp.where(xbuf[sl] >= thr_f, jnp.float32(1.0),
                             jnp.float32(0.0))

      out_d[r] = pltpu.async_copy(xbuf, out_hbm.at[base_row + r], semo)
      if r + 2 < rows_per_w:
        in_d[r + 2] = pltpu.async_copy(x_hbm.at[base_row + r + 2], cbuf,
                                       isems[(r + 2) % 3])
    out_d[rows_per_w - 1].wait()

  return pl.kernel(
      body,
      out_type=jax.ShapeDtypeStruct((batch, emb), jnp.float32),
      mesh=mesh,
      compiler_params=pltpu.CompilerParams(needs_layout_passes=False),
      scratch_types=[
          pltpu.VMEM((emb,), jnp.float32),
          pltpu.VMEM((emb,), jnp.float32),
          pltpu.VMEM((emb,), jnp.float32),
          pltpu.VMEM((NB * L,), jnp.int32),
          pltpu.SemaphoreType.DMA,
          pltpu.SemaphoreType.DMA,
          pltpu.SemaphoreType.DMA,
          pltpu.SemaphoreType.DMA,
      ],
  )


@jax.jit
def kernel(x):
  batch, emb = x.shape
  k = math.ceil(SPARSITY * emb)
  return _kwta_sc(batch, emb, k, 32)(x)
